# baseline (speedup 1.0000x reference)
import functools

import jax
import jax.numpy as jnp
from jax import lax
from jax.experimental import pallas as pl
from jax.experimental.pallas import tpu as pltpu
from jax.experimental.pallas import tpu_sc as plsc

_B, _S, _D, _H, _E, _K = 32, 4, 768, 256, 8, 2
_T = _B * _S
_EPS_NORM = 1e-6
_L = 16
_NCHUNK = _T // _L


def _router_logits_kernel(x_ref, nw_ref, gw_ref, lt_ref):
    x = x_ref[...]
    xn = x * lax.rsqrt(jnp.mean(x * x, axis=-1, keepdims=True) + _EPS_NORM)
    xn = xn * nw_ref[...]
    lt_ref[...] = lax.dot_general(
        gw_ref[...], xn, (((1,), (1,)), ((), ())),
        preferred_element_type=jnp.float32)


_NC = 2


def _sc_router_topk_kernel(lt_hbm, ct_hbm, lt_v, ct_v):
    wid = lax.axis_index("s") * _NC + lax.axis_index("c")

    @pl.when(wid < _NCHUNK)
    def _chunk():
        pltpu.sync_copy(lt_hbm, lt_v)
        base = wid * _L
        one = jnp.full((_L,), 1.0, jnp.float32)
        zero = jnp.zeros((_L,), jnp.float32)
        l = [lt_v[e, pl.ds(base, _L)] for e in range(_E)]
        m1 = l[0]
        for e in range(1, _E):
            m1 = jnp.maximum(m1, l[e])
        sel1, found = [], zero
        for e in range(_E):
            ge = jnp.where(l[e] >= m1, one, zero)
            s = ge * (one - found)
            sel1.append(s)
            found = found + s
        lm = [l[e] - sel1[e] * 1.0e30 for e in range(_E)]
        m2 = lm[0]
        for e in range(1, _E):
            m2 = jnp.maximum(m2, lm[e])
        sel2, found2 = [], zero
        for e in range(_E):
            ge = jnp.where(lm[e] >= m2, one, zero)
            s = ge * (one - found2)
            sel2.append(s)
            found2 = found2 + s
        z = zero
        for e in range(_E):
            z = z + jnp.exp(l[e] - m1)
        e2 = jnp.exp(m2 - m1)
        denom = 1.0 + e2 + 1e-10 * z
        c1 = 1.0 / denom
        c2 = e2 / denom
        for e in range(_E):
            ct_v[e, :] = sel1[e] * c1 + sel2[e] * c2
        pltpu.sync_copy(ct_v, ct_hbm.at[wid])


def _moe_ffn_kernel(x_ref, nw_ref, c_ref, w1_hbm, w2_hbm, w3_hbm, out_ref,
                    w1_buf, w2_buf, w3_buf, sems):
    def _copies(e):
        return (
            pltpu.make_async_copy(w1_hbm.at[e], w1_buf.at[e], sems.at[e, 0]),
            pltpu.make_async_copy(w2_hbm.at[e], w2_buf.at[e], sems.at[e, 1]),
            pltpu.make_async_copy(w3_hbm.at[e], w3_buf.at[e], sems.at[e, 2]),
        )

    for e in range(2):
        for cp in _copies(e):
            cp.start()

    x = x_ref[...]
    xn = x * lax.rsqrt(jnp.mean(x * x, axis=-1, keepdims=True) + _EPS_NORM)
    xn = xn * nw_ref[...]
    c = c_ref[...]
    lane = lax.broadcasted_iota(jnp.int32, c.shape, 1)

    acc = jnp.zeros(out_ref.shape, jnp.float32)
    for e in range(_E):
        if e + 2 < _E:
            for cp in _copies(e + 2):
                cp.start()
        for cp in _copies(e):
            cp.wait()
        xnb = xn.astype(jnp.bfloat16)
        h1 = jnp.dot(xnb, w1_buf[e].astype(jnp.bfloat16),
                     preferred_element_type=jnp.float32)
        h2 = jnp.dot(xnb, w2_buf[e].astype(jnp.bfloat16),
                     preferred_element_type=jnp.float32)
        hid = (h1 * lax.logistic(h1)) * h2
        oe = jnp.dot(hid.astype(jnp.bfloat16), w3_buf[e].astype(jnp.bfloat16),
                     preferred_element_type=jnp.float32)
        ce = jnp.sum(jnp.where(lane == e, c, 0.0), axis=-1, keepdims=True)
        acc = acc + ce * oe
    out_ref[...] = acc


def kernel(x, norm_weight, gate_w, w1, w2, w3):
    b, s, d = x.shape
    t = b * s
    x_flat = x.reshape(t, d)
    nw = norm_weight.reshape(1, d)

    lt = pl.pallas_call(
        _router_logits_kernel,
        out_shape=jax.ShapeDtypeStruct((_E, t), jnp.float32),
    )(x_flat, nw, gate_w)

    mesh = plsc.VectorSubcoreMesh(core_axis_name="c", subcore_axis_name="s")
    ct3 = functools.partial(
        pl.kernel, mesh=mesh,
        out_type=jax.ShapeDtypeStruct((_NCHUNK, _E, _L), jnp.float32),
        scratch_types=[
            pltpu.VMEM((_E, _T), jnp.float32),
            pltpu.VMEM((_E, _L), jnp.float32),
        ],
    )(_sc_router_topk_kernel)(lt)
    c_te = jnp.transpose(ct3, (0, 2, 1)).reshape(t, _E)

    out = pl.pallas_call(
        _moe_ffn_kernel,
        in_specs=[
            pl.BlockSpec((t, d), lambda: (0, 0)),
            pl.BlockSpec((1, d), lambda: (0, 0)),
            pl.BlockSpec((t, _E), lambda: (0, 0)),
            pl.BlockSpec(memory_space=pl.ANY),
            pl.BlockSpec(memory_space=pl.ANY),
            pl.BlockSpec(memory_space=pl.ANY),
        ],
        out_specs=pl.BlockSpec((t, d), lambda: (0, 0)),
        out_shape=jax.ShapeDtypeStruct((t, d), jnp.float32),
        scratch_shapes=[
            pltpu.VMEM((_E, _D, _H), jnp.float32),
            pltpu.VMEM((_E, _D, _H), jnp.float32),
            pltpu.VMEM((_E, _H, _D), jnp.float32),
            pltpu.SemaphoreType.DMA((_E, 3)),
        ],
    )(x_flat, nw, c_te, w1, w2, w3)
    return out.reshape(b, s, d)


# speedup vs baseline: 1.8576x; 1.8576x over previous
import jax
import jax.numpy as jnp
from jax.experimental import pallas as pl
from jax.experimental.pallas import tpu as pltpu

_B, _S, _D, _H, _E, _K = 32, 4, 768, 256, 8, 2
_EPS_NORM = 1e-6


def _moe_kernel(x_ref, nw_ref, gwt_ref, w1_hbm, w2_hbm, w3_hbm, out_ref,
                w1_buf, w2_buf, w3_buf, sems):
    _DH = _D // 2
    _HH = _H // 2

    def _copies(e):
        return (
            pltpu.make_async_copy(w1_hbm.at[e, pl.ds(0, _DH)],
                                  w1_buf.at[e, pl.ds(0, _DH)], sems.at[e, 0]),
            pltpu.make_async_copy(w2_hbm.at[e, pl.ds(0, _DH)],
                                  w2_buf.at[e, pl.ds(0, _DH)], sems.at[e, 1]),
            pltpu.make_async_copy(w1_hbm.at[e, pl.ds(_DH, _DH)],
                                  w1_buf.at[e, pl.ds(_DH, _DH)], sems.at[e, 2]),
            pltpu.make_async_copy(w2_hbm.at[e, pl.ds(_DH, _DH)],
                                  w2_buf.at[e, pl.ds(_DH, _DH)], sems.at[e, 3]),
            pltpu.make_async_copy(w3_hbm.at[e, pl.ds(0, _HH)],
                                  w3_buf.at[e, pl.ds(0, _HH)], sems.at[e, 4]),
            pltpu.make_async_copy(w3_hbm.at[e, pl.ds(_HH, _HH)],
                                  w3_buf.at[e, pl.ds(_HH, _HH)], sems.at[e, 5]),
        )

    for e in range(2):
        for cp in _copies(e):
            cp.start()

    x = x_ref[...]
    nw = nw_ref[...]
    xn = x * jax.lax.rsqrt(jnp.mean(x * x, axis=-1, keepdims=True) + _EPS_NORM)
    xn = xn * nw

    logits = jnp.dot(xn, gwt_ref[...], preferred_element_type=jnp.float32)
    p = jax.nn.softmax(logits, axis=-1)
    iota = jax.lax.broadcasted_iota(jnp.int32, p.shape, 1)
    m1 = jnp.max(p, axis=-1, keepdims=True)
    i1 = jnp.min(jnp.where(p >= m1, iota, _E), axis=-1, keepdims=True)
    one1 = iota == i1
    p2 = jnp.where(one1, -1.0, p)
    m2 = jnp.max(p2, axis=-1, keepdims=True)
    i2 = jnp.min(jnp.where(p2 >= m2, iota, _E), axis=-1, keepdims=True)
    one2 = iota == i2
    c = jnp.where(one1 | one2, p, 0.0) / (m1 + m2 + 1e-10)

    xnb = xn.astype(jnp.bfloat16)
    xna, xnz = xnb[:, :_DH], xnb[:, _DH:]
    acc = jnp.zeros(out_ref.shape, jnp.float32)
    for e in range(_E):
        if e + 2 < _E:
            for cp in _copies(e + 2):
                cp.start()
        cw1a, cw2a, cw1b, cw2b, cw3a, cw3b = _copies(e)
        cw1a.wait()
        h1 = jnp.dot(xna, w1_buf[e, :_DH].astype(jnp.bfloat16),
                     preferred_element_type=jnp.float32)
        cw2a.wait()
        h2 = jnp.dot(xna, w2_buf[e, :_DH].astype(jnp.bfloat16),
                     preferred_element_type=jnp.float32)
        cw1b.wait()
        h1 = h1 + jnp.dot(xnz, w1_buf[e, _DH:].astype(jnp.bfloat16),
                          preferred_element_type=jnp.float32)
        cw2b.wait()
        h2 = h2 + jnp.dot(xnz, w2_buf[e, _DH:].astype(jnp.bfloat16),
                          preferred_element_type=jnp.float32)
        hid = ((h1 * jax.lax.logistic(h1)) * h2).astype(jnp.bfloat16)
        cw3a.wait()
        oe = jnp.dot(hid[:, :_HH], w3_buf[e, :_HH].astype(jnp.bfloat16),
                     preferred_element_type=jnp.float32)
        cw3b.wait()
        oe = oe + jnp.dot(hid[:, _HH:], w3_buf[e, _HH:].astype(jnp.bfloat16),
                          preferred_element_type=jnp.float32)
        acc = acc + c[:, e:e + 1] * oe
    out_ref[...] = acc


def kernel(x, norm_weight, gate_w, w1, w2, w3):
    b, s, d = x.shape
    t = b * s
    x_flat = x.reshape(t, d)
    nw = norm_weight.reshape(1, d)
    gwt = gate_w.T
    out = pl.pallas_call(
        _moe_kernel,
        in_specs=[
            pl.BlockSpec((t, d), lambda: (0, 0)),
            pl.BlockSpec((1, d), lambda: (0, 0)),
            pl.BlockSpec((d, _E), lambda: (0, 0)),
            pl.BlockSpec(memory_space=pl.ANY),
            pl.BlockSpec(memory_space=pl.ANY),
            pl.BlockSpec(memory_space=pl.ANY),
        ],
        out_specs=pl.BlockSpec((t, d), lambda: (0, 0)),
        out_shape=jax.ShapeDtypeStruct((t, d), jnp.float32),
        scratch_shapes=[
            pltpu.VMEM((_E, _D, _H), jnp.float32),
            pltpu.VMEM((_E, _D, _H), jnp.float32),
            pltpu.VMEM((_E, _H, _D), jnp.float32),
            pltpu.SemaphoreType.DMA((_E, 6)),
        ],
    )(x_flat, nw, gwt, w1, w2, w3)
    return out.reshape(b, s, d)


# speedup vs baseline: 2.1770x; 1.1720x over previous
import jax
import jax.numpy as jnp
from jax.experimental import pallas as pl
from jax.experimental.pallas import tpu as pltpu

_B, _S, _D, _H, _E, _K = 32, 4, 768, 256, 8, 2
_EPS_NORM = 1e-6


def _moe_kernel(x_ref, nw_ref, gwt_ref, w1_hbm, w2_hbm, w3_hbm, out_ref,
                w1_buf, w2_buf, w3_buf, sems):
    def _copies(e):
        return (
            pltpu.make_async_copy(w1_hbm.at[e], w1_buf.at[e], sems.at[e, 0]),
            pltpu.make_async_copy(w2_hbm.at[e], w2_buf.at[e], sems.at[e, 1]),
            pltpu.make_async_copy(w3_hbm.at[e], w3_buf.at[e], sems.at[e, 2]),
        )

    for e in range(2):
        for cp in _copies(e):
            cp.start()

    x = x_ref[...]
    nw = nw_ref[...]
    xn = x * jax.lax.rsqrt(jnp.mean(x * x, axis=-1, keepdims=True) + _EPS_NORM)
    xn = xn * nw

    logits = jnp.dot(xn, gwt_ref[...], preferred_element_type=jnp.float32)
    p = jax.nn.softmax(logits, axis=-1)
    iota = jax.lax.broadcasted_iota(jnp.int32, p.shape, 1)
    m1 = jnp.max(p, axis=-1, keepdims=True)
    i1 = jnp.min(jnp.where(p >= m1, iota, _E), axis=-1, keepdims=True)
    one1 = iota == i1
    p2 = jnp.where(one1, -1.0, p)
    m2 = jnp.max(p2, axis=-1, keepdims=True)
    i2 = jnp.min(jnp.where(p2 >= m2, iota, _E), axis=-1, keepdims=True)
    one2 = iota == i2
    c = jnp.where(one1 | one2, p, 0.0) / (m1 + m2 + 1e-10)

    acc = jnp.zeros(out_ref.shape, jnp.float32)
    for e in range(_E):
        if e + 2 < _E:
            for cp in _copies(e + 2):
                cp.start()
        for cp in _copies(e):
            cp.wait()
        xnb = xn.astype(jnp.bfloat16)
        h1 = jnp.dot(xnb, w1_buf[e].astype(jnp.bfloat16),
                     preferred_element_type=jnp.float32)
        h2 = jnp.dot(xnb, w2_buf[e].astype(jnp.bfloat16),
                     preferred_element_type=jnp.float32)
        hid = (h1 * jax.lax.logistic(h1)) * h2
        oe = jnp.dot(hid.astype(jnp.bfloat16), w3_buf[e].astype(jnp.bfloat16),
                     preferred_element_type=jnp.float32)
        acc = acc + c[:, e:e + 1] * oe
    out_ref[...] = acc


def kernel(x, norm_weight, gate_w, w1, w2, w3):
    b, s, d = x.shape
    t = b * s
    x_flat = x.reshape(t, d)
    nw = norm_weight.reshape(1, d)
    gwt = gate_w.T
    out = pl.pallas_call(
        _moe_kernel,
        in_specs=[
            pl.BlockSpec((t, d), lambda: (0, 0)),
            pl.BlockSpec((1, d), lambda: (0, 0)),
            pl.BlockSpec((d, _E), lambda: (0, 0)),
            pl.BlockSpec(memory_space=pl.ANY),
            pl.BlockSpec(memory_space=pl.ANY),
            pl.BlockSpec(memory_space=pl.ANY),
        ],
        out_specs=pl.BlockSpec((t, d), lambda: (0, 0)),
        out_shape=jax.ShapeDtypeStruct((t, d), jnp.float32),
        scratch_shapes=[
            pltpu.VMEM((_E, _D, _H), jnp.float32),
            pltpu.VMEM((_E, _D, _H), jnp.float32),
            pltpu.VMEM((_E, _H, _D), jnp.float32),
            pltpu.SemaphoreType.DMA((_E, 3)),
        ],
    )(x_flat, nw, gwt, w1, w2, w3)
    return out.reshape(b, s, d)


# speedup vs baseline: 2.4714x; 1.1353x over previous
import jax
import jax.numpy as jnp
from jax.experimental import pallas as pl
from jax.experimental.pallas import tpu as pltpu

_B, _S, _D, _H, _E, _K = 32, 4, 768, 256, 8, 2
_EPS_NORM = 1e-6


def _moe_kernel(x_ref, nw_ref, gwt_ref, w1_hbm, w2_hbm, w3_hbm, out_ref,
                w1_buf, w2_buf, w3_buf, sems):
    def _copies(e):
        return (
            pltpu.make_async_copy(w1_hbm.at[e], w1_buf.at[e], sems.at[e, 0]),
            pltpu.make_async_copy(w2_hbm.at[e], w2_buf.at[e], sems.at[e, 1]),
            pltpu.make_async_copy(w3_hbm.at[e], w3_buf.at[e], sems.at[e, 2]),
        )

    for e in range(2):
        for cp in _copies(e):
            cp.start()

    x = x_ref[...]
    nw = nw_ref[...]
    xn = x * jax.lax.rsqrt(jnp.mean(x * x, axis=-1, keepdims=True) + _EPS_NORM)
    xn = xn * nw

    logits = jax.lax.dot_general(
        xn, gwt_ref[...], (((1,), (1,)), ((), ())),
        preferred_element_type=jnp.float32)
    p = jax.nn.softmax(logits, axis=-1)
    iota = jax.lax.broadcasted_iota(jnp.int32, p.shape, 1)
    m1 = jnp.max(p, axis=-1, keepdims=True)
    i1 = jnp.min(jnp.where(p >= m1, iota, _E), axis=-1, keepdims=True)
    one1 = iota == i1
    p2 = jnp.where(one1, -1.0, p)
    m2 = jnp.max(p2, axis=-1, keepdims=True)
    i2 = jnp.min(jnp.where(p2 >= m2, iota, _E), axis=-1, keepdims=True)
    one2 = iota == i2
    c = jnp.where(one1 | one2, p, 0.0) / (m1 + m2 + 1e-10)

    acc = jnp.zeros(out_ref.shape, jnp.float32)
    for e in range(_E):
        if e + 2 < _E:
            for cp in _copies(e + 2):
                cp.start()
        for cp in _copies(e):
            cp.wait()
        xnb = xn.astype(jnp.bfloat16)
        h1 = jnp.dot(xnb, w1_buf[e].astype(jnp.bfloat16),
                     preferred_element_type=jnp.float32)
        h2 = jnp.dot(xnb, w2_buf[e].astype(jnp.bfloat16),
                     preferred_element_type=jnp.float32)
        hid = (h1 * jax.lax.logistic(h1)) * h2
        oe = jnp.dot(hid.astype(jnp.bfloat16), w3_buf[e].astype(jnp.bfloat16),
                     preferred_element_type=jnp.float32)
        acc = acc + c[:, e:e + 1] * oe
    out_ref[...] = acc


def kernel(x, norm_weight, gate_w, w1, w2, w3):
    b, s, d = x.shape
    t = b * s
    x_flat = x.reshape(t, d)
    nw = norm_weight.reshape(1, d)
    out = pl.pallas_call(
        _moe_kernel,
        in_specs=[
            pl.BlockSpec((t, d), lambda: (0, 0)),
            pl.BlockSpec((1, d), lambda: (0, 0)),
            pl.BlockSpec((_E, d), lambda: (0, 0)),
            pl.BlockSpec(memory_space=pl.ANY),
            pl.BlockSpec(memory_space=pl.ANY),
            pl.BlockSpec(memory_space=pl.ANY),
        ],
        out_specs=pl.BlockSpec((t, d), lambda: (0, 0)),
        out_shape=jax.ShapeDtypeStruct((t, d), jnp.float32),
        scratch_shapes=[
            pltpu.VMEM((_E, _D, _H), jnp.float32),
            pltpu.VMEM((_E, _D, _H), jnp.float32),
            pltpu.VMEM((_E, _H, _D), jnp.float32),
            pltpu.SemaphoreType.DMA((_E, 3)),
        ],
    )(x_flat, nw, gate_w, w1, w2, w3)
    return out.reshape(b, s, d)
